# baseline (speedup 1.0000x reference)
import jax
import jax.numpy as jnp
from jax.experimental import pallas as pl
from jax.experimental.pallas import tpu as pltpu

B, C_IN, HF, WF = 8, 64, 128, 128
NUM_CLASSES = 80
K = 100
HW = HF * WF
NROWS = NUM_CLASSES * HF
NEG = -1e30
BIGI = 2**30


def _heads_kernel(x_ref, whm_ref, wrw_ref, znms_ref, rmax_ref, rw_ref):
    xb = x_ref[0]

    z = jnp.dot(whm_ref[...], xb, preferred_element_type=jnp.float32)
    rw = jnp.dot(wrw_ref[...], xb, preferred_element_type=jnp.float32)
    rw_ref[0] = rw.reshape(4 * HF, WF)

    z3 = z.reshape(NUM_CLASSES, HF, WF)
    negw = jnp.full((NUM_CLASSES, HF, 1), NEG, jnp.float32)
    zl = jnp.concatenate([z3[:, :, 1:], negw], axis=2)
    zr = jnp.concatenate([negw, z3[:, :, :-1]], axis=2)
    mw = jnp.maximum(jnp.maximum(zl, zr), z3)
    negh = jnp.full((NUM_CLASSES, 1, WF), NEG, jnp.float32)
    mu = jnp.concatenate([mw[:, 1:, :], negh], axis=1)
    md = jnp.concatenate([negh, mw[:, :-1, :]], axis=1)
    hmax = jnp.maximum(jnp.maximum(mu, md), mw)
    znms = jnp.where(hmax == z3, z3, NEG)

    znms_ref[0, :, 0, :] = znms.reshape(NROWS, WF)
    rmax_ref[0] = jnp.max(znms, axis=2)


def _select_kernel(rmax_ref, sel_ref):
    ridx = (jax.lax.broadcasted_iota(jnp.int32, (NUM_CLASSES, HF), 0) * HF
            + jax.lax.broadcasted_iota(jnp.int32, (NUM_CLASSES, HF), 1))
    lane = jax.lax.broadcasted_iota(jnp.int32, (1, WF), 1)

    def body(i, vals):
        new_vals = []
        row = jnp.zeros((1, WF), jnp.int32)
        for b in range(B):
            v = vals[b]
            m = jnp.max(v)
            r = jnp.min(jnp.where(v == m, ridx, BIGI))
            row = row + (lane == b) * r
            new_vals.append(jnp.where(ridx == r, -jnp.inf, v))
        sel_ref[pl.ds(i, 1), :] = row
        return tuple(new_vals)

    init = tuple(rmax_ref[b] for b in range(B))
    jax.lax.fori_loop(0, K, body, init, unroll=False)


def _gather_kernel(sel_ref, znms_ref, cand_ref):
    del sel_ref
    cand_ref[...] = znms_ref[...]


def _extract_kernel(sel_ref, cand_ref, rw_ref, out_ref, res_scr):
    lane = jax.lax.broadcasted_iota(jnp.int32, (1, WF), 1)
    lane100 = jax.lax.broadcasted_iota(jnp.int32, (K, WF), 1)
    sel = sel_ref[...]
    gidx = []
    for b in range(B):
        gidx.append(sel[:, b:b + 1] * WF + lane100)

    def body(j, vals):
        new_vals = []
        for b in range(B):
            v = vals[b]
            m = jnp.max(v)
            g = jnp.min(jnp.where(v == m, gidx[b], BIGI))
            c = g // HW
            sp = g - c * HW
            yy = sp // WF
            xx = sp - yy * WF
            onehot = (lane == xx).astype(jnp.float32)
            regx = jnp.sum(rw_ref[b, pl.ds(yy, 1), :] * onehot)
            regy = jnp.sum(rw_ref[b, pl.ds(HF + yy, 1), :] * onehot)
            ww = jnp.sum(rw_ref[b, pl.ds(2 * HF + yy, 1), :] * onehot)
            hh = jnp.sum(rw_ref[b, pl.ds(3 * HF + yy, 1), :] * onehot)
            score = jax.nn.sigmoid(m)
            xs = xx.astype(jnp.float32) + regx
            ys = yy.astype(jnp.float32) + regy
            row = ((lane == 0) * (xs - ww * 0.5) + (lane == 1) * (ys - hh * 0.5)
                   + (lane == 2) * (xs + ww * 0.5)
                   + (lane == 3) * (ys + hh * 0.5)
                   + (lane == 4) * score + (lane == 5) * c.astype(jnp.float32))
            res_scr[pl.ds(b * K + j, 1), :] = row.astype(jnp.float32)
            new_vals.append(jnp.where(gidx[b] == g, -jnp.inf, v))
        return tuple(new_vals)

    init = tuple(cand_ref[b, :, 0, :] for b in range(B))
    jax.lax.fori_loop(0, K, body, init, unroll=False)
    for b in range(B):
        out_ref[b, :, :] = res_scr[b * K:(b + 1) * K, :6]


@jax.jit
def kernel(x, W_hm, W_wh, W_reg):
    xf = x.reshape(B, C_IN, HW)
    wrw = jnp.concatenate([W_reg, W_wh], axis=0)

    znms, rmax, rw = pl.pallas_call(
        _heads_kernel,
        grid=(B,),
        in_specs=[
            pl.BlockSpec((1, C_IN, HW), lambda b: (b, 0, 0)),
            pl.BlockSpec((NUM_CLASSES, C_IN), lambda b: (0, 0)),
            pl.BlockSpec((4, C_IN), lambda b: (0, 0)),
        ],
        out_specs=[
            pl.BlockSpec((1, NROWS, 1, WF), lambda b: (b, 0, 0, 0)),
            pl.BlockSpec((1, NUM_CLASSES, HF), lambda b: (b, 0, 0)),
            pl.BlockSpec((1, 4 * HF, WF), lambda b: (b, 0, 0)),
        ],
        out_shape=[
            jax.ShapeDtypeStruct((B, NROWS, 1, WF), jnp.float32),
            jax.ShapeDtypeStruct((B, NUM_CLASSES, HF), jnp.float32),
            jax.ShapeDtypeStruct((B, 4 * HF, WF), jnp.float32),
        ],
        compiler_params=pltpu.CompilerParams(
            dimension_semantics=("parallel",),
        ),
    )(xf, W_hm, wrw)

    sel_wide = pl.pallas_call(
        _select_kernel,
        grid=(1,),
        in_specs=[pl.BlockSpec((B, NUM_CLASSES, HF), lambda i: (0, 0, 0))],
        out_specs=pl.BlockSpec((K, WF), lambda i: (0, 0)),
        out_shape=jax.ShapeDtypeStruct((K, WF), jnp.int32),
    )(rmax)
    sel = sel_wide[:, :B]

    cand = pl.pallas_call(
        _gather_kernel,
        grid_spec=pltpu.PrefetchScalarGridSpec(
            num_scalar_prefetch=1,
            grid=(B, K),
            in_specs=[
                pl.BlockSpec((1, 1, 1, WF), lambda b, i, s: (b, s[i, b], 0, 0)),
            ],
            out_specs=pl.BlockSpec((1, 1, 1, WF), lambda b, i, s: (b, i, 0, 0)),
        ),
        out_shape=jax.ShapeDtypeStruct((B, K, 1, WF), jnp.float32),
        compiler_params=pltpu.CompilerParams(
            dimension_semantics=("arbitrary", "arbitrary"),
        ),
    )(sel, znms)

    dets = pl.pallas_call(
        _extract_kernel,
        grid=(1,),
        in_specs=[
            pl.BlockSpec((K, B), lambda i: (0, 0)),
            pl.BlockSpec((B, K, 1, WF), lambda i: (0, 0, 0, 0)),
            pl.BlockSpec((B, 4 * HF, WF), lambda i: (0, 0, 0)),
        ],
        out_specs=pl.BlockSpec((B, K, 6), lambda i: (0, 0, 0)),
        out_shape=jax.ShapeDtypeStruct((B, K, 6), jnp.float32),
        scratch_shapes=[pltpu.VMEM((B * K, WF), jnp.float32)],
    )(sel, cand, rw)
    return dets


# speedup vs baseline: 1.3777x; 1.3777x over previous
import jax
import jax.numpy as jnp
from jax.experimental import pallas as pl
from jax.experimental.pallas import tpu as pltpu

B, C_IN, HF, WF = 8, 64, 128, 128
NUM_CLASSES = 80
K = 100
HW = HF * WF
NROWS = NUM_CLASSES * HF
NEG = -1e30
BIGI = 2**30
G = 2
PB = B // G


def _det_kernel(x_ref, whm_ref, wrw_ref, out_ref, hm_scr, rw_scr,
                cand_v, cand_g, res_scr):
    i = pl.program_id(1)

    @pl.when(i < PB)
    def _per_image():
        xb = x_ref[0]

        z = jnp.dot(whm_ref[...], xb, preferred_element_type=jnp.float32)
        rw = jnp.dot(wrw_ref[...], xb, preferred_element_type=jnp.float32)
        rw_scr[pl.ds(i * 4 * HF, 4 * HF), :] = rw.reshape(4 * HF, WF)

        z3 = z.reshape(NUM_CLASSES, HF, WF)
        negw = jnp.full((NUM_CLASSES, HF, 1), NEG, jnp.float32)
        zl = jnp.concatenate([z3[:, :, 1:], negw], axis=2)
        zr = jnp.concatenate([negw, z3[:, :, :-1]], axis=2)
        mw = jnp.maximum(jnp.maximum(zl, zr), z3)
        negh = jnp.full((NUM_CLASSES, 1, WF), NEG, jnp.float32)
        mu = jnp.concatenate([mw[:, 1:, :], negh], axis=1)
        md = jnp.concatenate([negh, mw[:, :-1, :]], axis=1)
        hmax = jnp.maximum(jnp.maximum(mu, md), mw)
        znms = jnp.where(hmax == z3, z3, NEG)

        hm_scr[...] = znms.reshape(NROWS, WF)
        rowmax = jnp.max(znms, axis=2).reshape(NUM_CLASSES, HF)

        ridx = (jax.lax.broadcasted_iota(jnp.int32, (NUM_CLASSES, HF), 0) * HF
                + jax.lax.broadcasted_iota(jnp.int32, (NUM_CLASSES, HF), 1))
        col = jax.lax.broadcasted_iota(jnp.int32, (1, WF), 1)

        def body_a(j, vals):
            m = jnp.max(vals)
            r = jnp.min(jnp.where(vals == m, ridx, BIGI))
            cand_v[pl.ds(i * K + j, 1), :] = hm_scr[pl.ds(r, 1), :]
            cand_g[pl.ds(i * K + j, 1), :] = r * WF + col
            return jnp.where(ridx == r, -jnp.inf, vals)

        jax.lax.fori_loop(0, K, body_a, rowmax, unroll=2)

    @pl.when(i == PB)
    def _extract():
        lane = jax.lax.broadcasted_iota(jnp.int32, (1, WF), 1)
        gidx = [cand_g[bb * K:(bb + 1) * K, :] for bb in range(PB)]

        def body_b(j, vals):
            new_vals = []
            for bb in range(PB):
                v = vals[bb]
                m = jnp.max(v)
                g = jnp.min(jnp.where(v == m, gidx[bb], BIGI))
                c = g // HW
                sp = g - c * HW
                yy = sp // WF
                xx = sp - yy * WF
                base = bb * 4 * HF
                onehot = (lane == xx).astype(jnp.float32)
                regx = jnp.sum(rw_scr[pl.ds(base + yy, 1), :] * onehot)
                regy = jnp.sum(rw_scr[pl.ds(base + HF + yy, 1), :] * onehot)
                ww = jnp.sum(rw_scr[pl.ds(base + 2 * HF + yy, 1), :] * onehot)
                hh = jnp.sum(rw_scr[pl.ds(base + 3 * HF + yy, 1), :] * onehot)
                score = jax.nn.sigmoid(m)
                xs = xx.astype(jnp.float32) + regx
                ys = yy.astype(jnp.float32) + regy
                row = ((lane == 0) * (xs - ww * 0.5)
                       + (lane == 1) * (ys - hh * 0.5)
                       + (lane == 2) * (xs + ww * 0.5)
                       + (lane == 3) * (ys + hh * 0.5)
                       + (lane == 4) * score
                       + (lane == 5) * c.astype(jnp.float32))
                res_scr[pl.ds(bb * K + j, 1), :] = row.astype(jnp.float32)
                new_vals.append(jnp.where(gidx[bb] == g, -jnp.inf, v))
            return tuple(new_vals)

        init = tuple(cand_v[bb * K:(bb + 1) * K, :] for bb in range(PB))
        jax.lax.fori_loop(0, K, body_b, init, unroll=False)
        for bb in range(PB):
            out_ref[bb, :, :] = res_scr[bb * K:(bb + 1) * K, :6]


@jax.jit
def kernel(x, W_hm, W_wh, W_reg):
    xf = x.reshape(B, C_IN, HW)
    wrw = jnp.concatenate([W_reg, W_wh], axis=0)
    dets = pl.pallas_call(
        _det_kernel,
        grid=(G, PB + 1),
        in_specs=[
            pl.BlockSpec((1, C_IN, HW),
                         lambda c, i: (c * PB + jnp.minimum(i, PB - 1), 0, 0)),
            pl.BlockSpec((NUM_CLASSES, C_IN), lambda c, i: (0, 0)),
            pl.BlockSpec((4, C_IN), lambda c, i: (0, 0)),
        ],
        out_specs=pl.BlockSpec((PB, K, 6), lambda c, i: (c, 0, 0)),
        out_shape=jax.ShapeDtypeStruct((B, K, 6), jnp.float32),
        scratch_shapes=[
            pltpu.VMEM((NROWS, WF), jnp.float32),
            pltpu.VMEM((PB * 4 * HF, WF), jnp.float32),
            pltpu.VMEM((PB * K, WF), jnp.float32),
            pltpu.VMEM((PB * K, WF), jnp.int32),
            pltpu.VMEM((PB * K, WF), jnp.float32),
        ],
        compiler_params=pltpu.CompilerParams(
            dimension_semantics=("parallel", "arbitrary"),
        ),
    )(xf, W_hm, wrw)
    return dets
